# baseline (speedup 1.0000x reference)
import numpy as np
import jax
import jax.numpy as jnp
from jax.experimental import pallas as pl
from jax.experimental.pallas import tpu as pltpu

BB = 128
TILES = 4
BSTEP = BB * TILES
f32 = jnp.float32
bf16 = jnp.bfloat16


def _sel(nvo, npo, nvi, npi, off=0):
    e = np.zeros((npo, 3, npi), np.float32)
    for p in range(nvo):
        for k in range(3):
            s = p + k - 1 + off
            if 0 <= s < nvi:
                e[p, k, s] = 1.0
    return e


_EW1G = _sel(28, 32, 28, 28).reshape(16, 2, 3, 28).transpose(1, 0, 2, 3)
_ETOPG = _sel(4, 4, 7, 7, off=0).reshape(2, 2, 3, 7).transpose(1, 0, 2, 3)
_EMIDG = _sel(4, 4, 7, 7, off=2).reshape(2, 2, 3, 7).transpose(1, 0, 2, 3)
_EW2G = _sel(14, 16, 14, 16).reshape(8, 2, 3, 16).transpose(1, 0, 2, 3)
_ED2 = _sel(2, 2, 4, 4, off=1)
_E1G = np.stack([_ETOPG, _EMIDG])


def _body(x_ref, w1_ref, nb1_ref, w2_ref, nb2_ref, wfc_ref, bfc_ref,
          out_ref, x1p, x2p, feat):
    x1p[pl.ds(784, 32), :] = jnp.zeros((32, 256), bf16)
    x2p[:, pl.ds(0, 1)] = jnp.zeros((9, 1, 16, 256), bf16)
    x2p[:, pl.ds(15, 1)] = jnp.zeros((9, 1, 16, 256), bf16)
    x2p[pl.ds(8, 1), pl.ds(1, 14)] = jnp.zeros((1, 14, 16, 256), bf16)
    x2p[pl.ds(8, 1), pl.ds(1, 14), pl.ds(0, 14)] = jnp.ones((1, 14, 14, 256),
                                                            bf16)
    feat[:, pl.ds(7, 1)] = jnp.zeros((16, 1, 8, 256), bf16)

    for pair in range(TILES // 2):
        for h in range(2):
            xt = jnp.transpose(
                x_ref[pl.ds((2 * pair + h) * BB, BB), :, :].astype(bf16)
                .reshape(BB, 784))
            x1p[pl.ds(0, 784), pl.ds(h * BB, BB)] = xt

        for r in range(7):
            wof = 0 if r == 0 else 1024
            src = x1p[pl.ds(max(4 * r - 2, 0) * 28, 196), :]
            a0 = jnp.dot(w1_ref[pl.ds(wof + 0, 256), :], src,
                         preferred_element_type=f32)
            a1 = jnp.dot(w1_ref[pl.ds(wof + 256, 256), :], src,
                         preferred_element_type=f32)
            a2 = jnp.dot(w1_ref[pl.ds(wof + 512, 256), :], src,
                         preferred_element_type=f32)
            a3 = jnp.dot(w1_ref[pl.ds(wof + 768, 256), :], src,
                         preferred_element_type=f32)
            pooled = jnp.maximum(jnp.maximum(jnp.maximum(a0, a1),
                                             jnp.maximum(a2, a3)),
                                 nb1_ref[...])
            pb = pooled.astype(bf16)
            for h in range(2):
                x2p[pl.ds(0, 8), pl.ds(2 * r + 1, 2), :, pl.ds(h * BB, BB)] = (
                    pb[:, h * BB:(h + 1) * BB].reshape(8, 2, 16, BB))

        for s in range(7):
            src = x2p[:, pl.ds(2 * s, 4)].reshape(576, 256)
            a0 = jnp.dot(w2_ref[pl.ds(0, 128), :], src,
                         preferred_element_type=f32)
            a1 = jnp.dot(w2_ref[pl.ds(128, 128), :], src,
                         preferred_element_type=f32)
            a2 = jnp.dot(w2_ref[pl.ds(256, 128), :], src,
                         preferred_element_type=f32)
            a3 = jnp.dot(w2_ref[pl.ds(384, 128), :], src,
                         preferred_element_type=f32)
            pooled = jnp.maximum(jnp.maximum(jnp.maximum(a0, a1),
                                             jnp.maximum(a2, a3)),
                                 nb2_ref[...])
            for h in range(2):
                feat[:, pl.ds(s, 1), :, pl.ds(h * BB, BB)] = (
                    pooled[:, h * BB:(h + 1) * BB].astype(bf16)
                    .reshape(16, 1, 8, BB))

        logits = jnp.dot(wfc_ref[...], feat[...].reshape(1024, 256),
                         preferred_element_type=f32)
        out_ref[:, pl.ds(pair * 256, 256)] = logits + bfc_ref[...]


def kernel(x, w1s, b1, w2s, b2, wfc_pad, bfc):
    N = x.shape[0]
    w1r = w1s.astype(f32).reshape(8, 3, 3)
    w2r = w2s.astype(f32).reshape(16, 8, 3, 3)
    b1f = b1.astype(f32)
    b2f = b2.astype(f32)
    wfc = wfc_pad.astype(f32)
    w1both = jnp.einsum('okl,mcdkp,ewlq->mceodwpq', w1r, _E1G, _EW1G)
    w1both = w1both.reshape(2048, 196).astype(bf16)
    btap = jnp.einsum('oikl,i->okl', w2r, b1f)
    w2aug = jnp.concatenate([w2r, btap[:, None]], axis=1)
    w2full = jnp.einsum('oikl,dkj,ewlq->deowijq', w2aug, _ED2, _EW2G)
    w2full = w2full.reshape(512, 576).astype(bf16)
    nb1 = jnp.repeat(-b1f, 32).reshape(256, 1)
    nb2 = jnp.repeat(-b2f, 8).reshape(128, 1)
    bfc2 = bfc.astype(f32) + (wfc @ jnp.repeat(b2f, 64)).reshape(10, 1)

    n_tiles = (N + BSTEP - 1) // BSTEP
    npad = n_tiles * BSTEP
    xt = x.reshape(N, 28, 28)
    if npad != N:
        xt = jnp.pad(xt, ((0, npad - N), (0, 0), (0, 0)))

    flops = 2 * npad * (7 * 1024 * 196 + 7 * 512 * 576 + 1024 * 10)
    bytes_accessed = 4 * xt.size + 2 * 2 * 1024 * 196 + 2 * 512 * 576 \
        + 4 * npad * 10

    out = pl.pallas_call(
        _body,
        out_shape=jax.ShapeDtypeStruct((10, npad), f32),
        grid_spec=pltpu.PrefetchScalarGridSpec(
            num_scalar_prefetch=0,
            grid=(n_tiles,),
            in_specs=[
                pl.BlockSpec((BSTEP, 28, 28), lambda i: (i, 0, 0)),
                pl.BlockSpec((2048, 196), lambda i: (0, 0)),
                pl.BlockSpec((256, 1), lambda i: (0, 0)),
                pl.BlockSpec((512, 576), lambda i: (0, 0)),
                pl.BlockSpec((128, 1), lambda i: (0, 0)),
                pl.BlockSpec((10, 1024), lambda i: (0, 0)),
                pl.BlockSpec((10, 1), lambda i: (0, 0)),
            ],
            out_specs=pl.BlockSpec((10, BSTEP), lambda i: (0, i)),
            scratch_shapes=[
                pltpu.VMEM((816, 256), bf16),
                pltpu.VMEM((9, 16, 16, 256), bf16),
                pltpu.VMEM((16, 8, 8, 256), bf16),
            ]),
        compiler_params=pltpu.CompilerParams(
            dimension_semantics=("parallel",),
            vmem_limit_bytes=32 * 1024 * 1024),
        cost_estimate=pl.CostEstimate(flops=flops, transcendentals=0,
                                      bytes_accessed=bytes_accessed),
    )(xt, w1both, nb1, w2full, nb2, wfc, bfc2)

    return out[:, :N].T
